# Initial kernel scaffold; baseline (speedup 1.0000x reference)
import jax
import jax.numpy as jnp
from jax.experimental import pallas as pl


def kernel(x, edge_index, W1, b1, W2, b2):
    raise NotImplementedError("write your pallas kernel here")


# speedup vs baseline: 27.7380x; 27.7380x over previous
import functools

import jax
import jax.numpy as jnp
from jax import lax
from jax.experimental import pallas as pl
from jax.experimental.pallas import tpu as pltpu
from jax.experimental.pallas import tpu_sc as plsc

NC = 2
NS = 16
NW = NC * NS
LANES = 128


def _sc_mesh():
    return plsc.VectorSubcoreMesh(core_axis_name="c", subcore_axis_name="s")


_SC_PARAMS = pltpu.CompilerParams(use_tc_tiling_on_sc=False)


def _deg_sc(rows2d, ones_hbm, zeros_hbm, n_pad, chunks_w):
    blk = n_pad // NS

    @functools.partial(
        pl.kernel,
        out_type=jax.ShapeDtypeStruct((NC, n_pad), jnp.float32),
        mesh=_sc_mesh(),
        compiler_params=_SC_PARAMS,
        scratch_types=[
            pltpu.VMEM((chunks_w, LANES), jnp.int32),
            pltpu.VMEM((LANES,), jnp.float32),
            pltpu.VMEM((blk,), jnp.float32),
            pltpu.VMEM_SHARED((n_pad,), jnp.float32),
        ],
    )
    def deg_kernel(r_hbm, one_hbm, z_hbm, out_hbm, ridx, ones_v, tmp, acc):
        c = lax.axis_index("c")
        s = lax.axis_index("s")
        w = s * NC + c
        pltpu.sync_copy(z_hbm.at[pl.ds(s * blk, blk)], tmp)
        pltpu.sync_copy(tmp, acc.at[pl.ds(s * blk, blk)])
        pltpu.sync_copy(one_hbm, ones_v)
        pltpu.sync_copy(r_hbm.at[pl.ds(w * chunks_w, chunks_w)], ridx)
        plsc.subcore_barrier()

        def body(j, carry):
            pltpu.sync_copy(ones_v, acc.at[ridx.at[j]], add=True)
            return carry

        lax.fori_loop(0, chunks_w, body, 0)
        plsc.subcore_barrier()
        pltpu.sync_copy(acc.at[pl.ds(s * blk, blk)], tmp)
        pltpu.sync_copy(tmp, out_hbm.at[c, pl.ds(s * blk, blk)])

    return deg_kernel(rows2d, ones_hbm, zeros_hbm)


def _spmm_sc(h, rows2d, cols2d, zeros_hbm, n_pad, feat, chunks_w):
    blk = n_pad // NS

    @functools.partial(
        pl.kernel,
        out_type=jax.ShapeDtypeStruct((NC, n_pad, feat), jnp.float32),
        mesh=_sc_mesh(),
        compiler_params=_SC_PARAMS,
        scratch_types=[
            pltpu.VMEM((chunks_w, LANES), jnp.int32),
            pltpu.VMEM((chunks_w, LANES), jnp.int32),
            pltpu.VMEM((LANES, feat), jnp.float32),
            pltpu.VMEM((blk, feat), jnp.float32),
            pltpu.VMEM_SHARED((n_pad, feat), jnp.float32),
            pltpu.SemaphoreType.DMA,
        ],
    )
    def spmm_kernel(h_hbm, r_hbm, c_hbm, z_hbm, out_hbm,
                    ridx, cidx, msg, tmp, acc, sem):
        c = lax.axis_index("c")
        s = lax.axis_index("s")
        w = s * NC + c
        pltpu.sync_copy(z_hbm.at[pl.ds(s * blk, blk)], tmp)
        pltpu.sync_copy(tmp, acc.at[pl.ds(s * blk, blk)])
        pltpu.sync_copy(r_hbm.at[pl.ds(w * chunks_w, chunks_w)], ridx)
        pltpu.sync_copy(c_hbm.at[pl.ds(w * chunks_w, chunks_w)], cidx)
        plsc.subcore_barrier()

        def body(j, carry):
            pltpu.async_copy(h_hbm.at[ridx.at[j]], msg, sem).wait()
            pltpu.sync_copy(msg, acc.at[cidx.at[j]], add=True)
            return carry

        lax.fori_loop(0, chunks_w, body, 0)
        plsc.subcore_barrier()
        pltpu.sync_copy(acc.at[pl.ds(s * blk, blk)], tmp)
        pltpu.sync_copy(tmp, out_hbm.at[c, pl.ds(s * blk, blk)])

    return spmm_kernel(h, rows2d, cols2d, zeros_hbm)


def _tc_layer1(deg0, deg1, xp, w1t, b1p, n_pad, f1):
    blkn = 1024
    d = xp.shape[1]

    def body(d0_ref, d1_ref, x_ref, w_ref, b_ref, h_ref, di_ref):
        di = lax.rsqrt(d0_ref[...] + d1_ref[...])
        h = jnp.dot(x_ref[...], w_ref[...], preferred_element_type=jnp.float32)
        h_ref[...] = (h + b_ref[...]) * di
        di_ref[...] = di

    return pl.pallas_call(
        body,
        grid=(pl.cdiv(n_pad, blkn),),
        in_specs=[
            pl.BlockSpec((blkn, 1), lambda i: (i, 0)),
            pl.BlockSpec((blkn, 1), lambda i: (i, 0)),
            pl.BlockSpec((blkn, d), lambda i: (i, 0)),
            pl.BlockSpec((d, f1), lambda i: (0, 0)),
            pl.BlockSpec((1, f1), lambda i: (0, 0)),
        ],
        out_specs=[
            pl.BlockSpec((blkn, f1), lambda i: (i, 0)),
            pl.BlockSpec((blkn, 1), lambda i: (i, 0)),
        ],
        out_shape=[
            jax.ShapeDtypeStruct((n_pad, f1), jnp.float32),
            jax.ShapeDtypeStruct((n_pad, 1), jnp.float32),
        ],
    )(deg0, deg1, xp, w1t, b1p)


def _tc_layer2(s1a, s1b, dinv, w2t, b2p, n_pad, f1, f2):
    blkn = 1024

    def body(a_ref, b_ref, di_ref, w_ref, bias_ref, out_ref):
        di = di_ref[...]
        h = jnp.maximum((a_ref[...] + b_ref[...]) * di, 0.0)
        out = jnp.dot(h, w_ref[...], preferred_element_type=jnp.float32)
        out_ref[...] = (out + bias_ref[...]) * di

    return pl.pallas_call(
        body,
        grid=(pl.cdiv(n_pad, blkn),),
        in_specs=[
            pl.BlockSpec((blkn, f1), lambda i: (i, 0)),
            pl.BlockSpec((blkn, f1), lambda i: (i, 0)),
            pl.BlockSpec((blkn, 1), lambda i: (i, 0)),
            pl.BlockSpec((f1, f2), lambda i: (0, 0)),
            pl.BlockSpec((1, f2), lambda i: (0, 0)),
        ],
        out_specs=pl.BlockSpec((blkn, f2), lambda i: (i, 0)),
        out_shape=jax.ShapeDtypeStruct((n_pad, f2), jnp.float32),
    )(s1a, s1b, dinv, w2t, b2p)


def _tc_logsoftmax(s2a, s2b, dinv, n_pad, f2, ncls):
    blkn = 1024

    def body(a_ref, b_ref, di_ref, out_ref):
        z = ((a_ref[...] + b_ref[...]) * di_ref[...])[:, :ncls]
        m = jnp.max(z, axis=1, keepdims=True)
        e = jnp.exp(z - m)
        lse = jnp.log(jnp.sum(e, axis=1, keepdims=True)) + m
        out_ref[...] = z - lse

    return pl.pallas_call(
        body,
        grid=(pl.cdiv(n_pad, blkn),),
        in_specs=[
            pl.BlockSpec((blkn, f2), lambda i: (i, 0)),
            pl.BlockSpec((blkn, f2), lambda i: (i, 0)),
            pl.BlockSpec((blkn, 1), lambda i: (i, 0)),
        ],
        out_specs=pl.BlockSpec((blkn, ncls), lambda i: (i, 0)),
        out_shape=jax.ShapeDtypeStruct((n_pad, ncls), jnp.float32),
    )(s2a, s2b, dinv)


def kernel(x, edge_index, W1, b1, W2, b2):
    n, d = x.shape
    hid = W1.shape[0]
    ncls = W2.shape[0]
    e_tot = edge_index.shape[1] + n
    f1 = 16

    chunks_w = -(-e_tot // (LANES * NW))
    chunks_w = -(-chunks_w // 8) * 8
    e_pad = chunks_w * LANES * NW
    n_pad = -(-(n + 1) // LANES) * LANES

    ei = edge_index.astype(jnp.int32)
    loops = jnp.arange(n, dtype=jnp.int32)
    pad_idx = n + jnp.arange(e_pad - e_tot, dtype=jnp.int32) % (n_pad - n)
    rows = jnp.concatenate([ei[0], loops, pad_idx]).reshape(-1, LANES)
    cols = jnp.concatenate([ei[1], loops, pad_idx]).reshape(-1, LANES)

    f2 = -(-ncls // 16) * 16
    ones_l = jnp.ones((LANES,), jnp.float32)
    z1 = jnp.zeros((n_pad,), jnp.float32)
    zf1 = jnp.zeros((n_pad, f1), jnp.float32)
    zf2 = jnp.zeros((n_pad, f2), jnp.float32)
    xp = jnp.pad(x, ((0, n_pad - n), (0, 0)))
    w1t = jnp.pad(W1.T, ((0, 0), (0, f1 - hid)))
    b1p = jnp.pad(b1, (0, f1 - hid)).reshape(1, f1)
    w2t = jnp.pad(W2.T, ((0, f1 - hid), (0, f2 - ncls)))
    b2p = jnp.pad(b2, (0, f2 - ncls)).reshape(1, f2)

    dega, degb = _deg_sc(rows, ones_l, z1, n_pad, chunks_w)
    deg0 = dega.reshape(n_pad, 1)
    deg1 = degb.reshape(n_pad, 1)

    h1p, dinv = _tc_layer1(deg0, deg1, xp, w1t, b1p, n_pad, f1)
    s1 = _spmm_sc(h1p, rows, cols, zf1, n_pad, f1, chunks_w)
    h2p = _tc_layer2(s1[0], s1[1], dinv, w2t, b2p, n_pad, f1, f2)
    s2 = _spmm_sc(h2p, rows, cols, zf2, n_pad, f2, chunks_w)
    out = _tc_logsoftmax(s2[0], s2[1], dinv, n_pad, f2, ncls)
    return out[:n]


# speedup vs baseline: 41.1026x; 1.4818x over previous
import functools

import jax
import jax.numpy as jnp
from jax import lax
from jax.experimental import pallas as pl
from jax.experimental.pallas import tpu as pltpu
from jax.experimental.pallas import tpu_sc as plsc

NC = 2
NS = 16
NW = NC * NS
LANES = 128


def _sc_mesh():
    return plsc.VectorSubcoreMesh(core_axis_name="c", subcore_axis_name="s")


_SC_PARAMS = pltpu.CompilerParams(use_tc_tiling_on_sc=False)


def _deg_sc(rows2d, ones_hbm, zeros_hbm, n_pad, chunks_w):
    blk = n_pad // NS

    @functools.partial(
        pl.kernel,
        out_type=jax.ShapeDtypeStruct((NC, n_pad), jnp.float32),
        mesh=_sc_mesh(),
        compiler_params=_SC_PARAMS,
        scratch_types=[
            pltpu.VMEM((chunks_w, LANES), jnp.int32),
            pltpu.VMEM((LANES,), jnp.float32),
            pltpu.VMEM((blk,), jnp.float32),
            pltpu.VMEM_SHARED((n_pad,), jnp.float32),
        ],
    )
    def deg_kernel(r_hbm, one_hbm, z_hbm, out_hbm, ridx, ones_v, tmp, acc):
        c = lax.axis_index("c")
        s = lax.axis_index("s")
        w = s * NC + c
        pltpu.sync_copy(z_hbm.at[pl.ds(s * blk, blk)], tmp)
        pltpu.sync_copy(tmp, acc.at[pl.ds(s * blk, blk)])
        pltpu.sync_copy(one_hbm, ones_v)
        pltpu.sync_copy(r_hbm.at[pl.ds(w * chunks_w, chunks_w)], ridx)
        plsc.subcore_barrier()

        def body(j, carry):
            pltpu.sync_copy(ones_v, acc.at[ridx.at[j]], add=True)
            return carry

        lax.fori_loop(0, chunks_w, body, 0)
        plsc.subcore_barrier()
        pltpu.sync_copy(acc.at[pl.ds(s * blk, blk)], tmp)
        pltpu.sync_copy(tmp, out_hbm.at[c, pl.ds(s * blk, blk)])

    return deg_kernel(rows2d, ones_hbm, zeros_hbm)


NBUF = 4


def _spmm_sc(h, rows2d, cols2d, zeros_hbm, n_pad, feat, chunks_w):
    blk = n_pad // NS
    assert chunks_w % NBUF == 0

    @functools.partial(
        pl.kernel,
        out_type=jax.ShapeDtypeStruct((NC, n_pad, feat), jnp.float32),
        mesh=_sc_mesh(),
        compiler_params=_SC_PARAMS,
        scratch_types=[
            pltpu.VMEM((chunks_w + NBUF, LANES), jnp.int32),
            pltpu.VMEM((chunks_w, LANES), jnp.int32),
            [pltpu.VMEM((LANES, feat), jnp.float32) for _ in range(NBUF)],
            pltpu.VMEM((blk, feat), jnp.float32),
            pltpu.VMEM_SHARED((n_pad, feat), jnp.float32),
            [pltpu.SemaphoreType.DMA for _ in range(NBUF)],
        ],
    )
    def spmm_kernel(h_hbm, r_hbm, c_hbm, z_hbm, out_hbm,
                    ridx, cidx, msgs, tmp, acc, sems):
        c = lax.axis_index("c")
        s = lax.axis_index("s")
        w = s * NC + c
        pltpu.sync_copy(z_hbm.at[pl.ds(s * blk, blk)], tmp)
        pltpu.sync_copy(tmp, acc.at[pl.ds(s * blk, blk)])
        pltpu.sync_copy(
            r_hbm.at[pl.ds(w * chunks_w, chunks_w)], ridx.at[pl.ds(0, chunks_w)]
        )
        pltpu.sync_copy(
            r_hbm.at[pl.ds(w * chunks_w, NBUF)], ridx.at[pl.ds(chunks_w, NBUF)]
        )
        pltpu.sync_copy(c_hbm.at[pl.ds(w * chunks_w, chunks_w)], cidx)
        plsc.subcore_barrier()

        for b in range(NBUF):
            pltpu.async_copy(h_hbm.at[ridx.at[b]], msgs[b], sems[b])

        def body(t, carry):
            for b in range(NBUF):
                j = t * NBUF + b
                pltpu.make_async_copy(h_hbm.at[ridx.at[j]], msgs[b], sems[b]).wait()
                pltpu.sync_copy(msgs[b], acc.at[cidx.at[j]], add=True)
                pltpu.async_copy(h_hbm.at[ridx.at[j + NBUF]], msgs[b], sems[b])
            return carry

        lax.fori_loop(0, chunks_w // NBUF, body, 0)
        for b in range(NBUF):
            pltpu.make_async_copy(h_hbm.at[ridx.at[b]], msgs[b], sems[b]).wait()
        plsc.subcore_barrier()
        pltpu.sync_copy(acc.at[pl.ds(s * blk, blk)], tmp)
        pltpu.sync_copy(tmp, out_hbm.at[c, pl.ds(s * blk, blk)])

    return spmm_kernel(h, rows2d, cols2d, zeros_hbm)


def _tc_layer1(deg0, deg1, xp, w1t, b1p, n_pad, f1):
    blkn = 1024
    d = xp.shape[1]

    def body(d0_ref, d1_ref, x_ref, w_ref, b_ref, h_ref, di_ref):
        di = lax.rsqrt(d0_ref[...] + d1_ref[...])
        h = jnp.dot(x_ref[...], w_ref[...], preferred_element_type=jnp.float32)
        h_ref[...] = (h + b_ref[...]) * di
        di_ref[...] = di

    return pl.pallas_call(
        body,
        grid=(pl.cdiv(n_pad, blkn),),
        in_specs=[
            pl.BlockSpec((blkn, 1), lambda i: (i, 0)),
            pl.BlockSpec((blkn, 1), lambda i: (i, 0)),
            pl.BlockSpec((blkn, d), lambda i: (i, 0)),
            pl.BlockSpec((d, f1), lambda i: (0, 0)),
            pl.BlockSpec((1, f1), lambda i: (0, 0)),
        ],
        out_specs=[
            pl.BlockSpec((blkn, f1), lambda i: (i, 0)),
            pl.BlockSpec((blkn, 1), lambda i: (i, 0)),
        ],
        out_shape=[
            jax.ShapeDtypeStruct((n_pad, f1), jnp.float32),
            jax.ShapeDtypeStruct((n_pad, 1), jnp.float32),
        ],
    )(deg0, deg1, xp, w1t, b1p)


def _tc_layer2(s1a, s1b, dinv, w2t, b2p, n_pad, f1, f2):
    blkn = 1024

    def body(a_ref, b_ref, di_ref, w_ref, bias_ref, out_ref):
        di = di_ref[...]
        h = jnp.maximum((a_ref[...] + b_ref[...]) * di, 0.0)
        out = jnp.dot(h, w_ref[...], preferred_element_type=jnp.float32)
        out_ref[...] = (out + bias_ref[...]) * di

    return pl.pallas_call(
        body,
        grid=(pl.cdiv(n_pad, blkn),),
        in_specs=[
            pl.BlockSpec((blkn, f1), lambda i: (i, 0)),
            pl.BlockSpec((blkn, f1), lambda i: (i, 0)),
            pl.BlockSpec((blkn, 1), lambda i: (i, 0)),
            pl.BlockSpec((f1, f2), lambda i: (0, 0)),
            pl.BlockSpec((1, f2), lambda i: (0, 0)),
        ],
        out_specs=pl.BlockSpec((blkn, f2), lambda i: (i, 0)),
        out_shape=jax.ShapeDtypeStruct((n_pad, f2), jnp.float32),
    )(s1a, s1b, dinv, w2t, b2p)


def _tc_logsoftmax(s2a, s2b, dinv, n_pad, f2, ncls):
    blkn = 1024

    def body(a_ref, b_ref, di_ref, out_ref):
        z = ((a_ref[...] + b_ref[...]) * di_ref[...])[:, :ncls]
        m = jnp.max(z, axis=1, keepdims=True)
        e = jnp.exp(z - m)
        lse = jnp.log(jnp.sum(e, axis=1, keepdims=True)) + m
        out_ref[...] = z - lse

    return pl.pallas_call(
        body,
        grid=(pl.cdiv(n_pad, blkn),),
        in_specs=[
            pl.BlockSpec((blkn, f2), lambda i: (i, 0)),
            pl.BlockSpec((blkn, f2), lambda i: (i, 0)),
            pl.BlockSpec((blkn, 1), lambda i: (i, 0)),
        ],
        out_specs=pl.BlockSpec((blkn, ncls), lambda i: (i, 0)),
        out_shape=jax.ShapeDtypeStruct((n_pad, ncls), jnp.float32),
    )(s2a, s2b, dinv)


def kernel(x, edge_index, W1, b1, W2, b2):
    n, d = x.shape
    hid = W1.shape[0]
    ncls = W2.shape[0]
    e_tot = edge_index.shape[1] + n
    f1 = 16

    chunks_w = -(-e_tot // (LANES * NW))
    chunks_w = -(-chunks_w // 8) * 8
    e_pad = chunks_w * LANES * NW
    n_pad = -(-(n + 1) // LANES) * LANES

    ei = edge_index.astype(jnp.int32)
    loops = jnp.arange(n, dtype=jnp.int32)
    pad_idx = n + jnp.arange(e_pad - e_tot, dtype=jnp.int32) % (n_pad - n)
    rows = jnp.concatenate([ei[0], loops, pad_idx]).reshape(-1, LANES)
    cols = jnp.concatenate([ei[1], loops, pad_idx]).reshape(-1, LANES)

    f2 = -(-ncls // 8) * 8
    ones_l = jnp.ones((LANES,), jnp.float32)
    z1 = jnp.zeros((n_pad,), jnp.float32)
    zf1 = jnp.zeros((n_pad, f1), jnp.float32)
    zf2 = jnp.zeros((n_pad, f2), jnp.float32)
    xp = jnp.pad(x, ((0, n_pad - n), (0, 0)))
    w1t = jnp.pad(W1.T, ((0, 0), (0, f1 - hid)))
    b1p = jnp.pad(b1, (0, f1 - hid)).reshape(1, f1)
    w2t = jnp.pad(W2.T, ((0, f1 - hid), (0, f2 - ncls)))
    b2p = jnp.pad(b2, (0, f2 - ncls)).reshape(1, f2)

    dega, degb = _deg_sc(rows, ones_l, z1, n_pad, chunks_w)
    deg0 = dega.reshape(n_pad, 1)
    deg1 = degb.reshape(n_pad, 1)

    h1p, dinv = _tc_layer1(deg0, deg1, xp, w1t, b1p, n_pad, f1)
    s1 = _spmm_sc(h1p, rows, cols, zf1, n_pad, f1, chunks_w)
    h2p = _tc_layer2(s1[0], s1[1], dinv, w2t, b2p, n_pad, f1, f2)
    s2 = _spmm_sc(h2p, rows, cols, zf2, n_pad, f2, chunks_w)
    out = _tc_logsoftmax(s2[0], s2[1], dinv, n_pad, f2, ncls)
    return out[:n]


# speedup vs baseline: 45.1122x; 1.0976x over previous
import functools

import jax
import jax.numpy as jnp
import numpy as np
from jax import lax
from jax.experimental import pallas as pl
from jax.experimental.pallas import tpu as pltpu
from jax.experimental.pallas import tpu_sc as plsc

NC = 2
NS = 16
NW = NC * NS
LANES = 128
NBUF = 8


def _sc_mesh():
    return plsc.VectorSubcoreMesh(core_axis_name="c", subcore_axis_name="s")


_SC_PARAMS = pltpu.CompilerParams(use_tc_tiling_on_sc=False)


def _deg_sc(rows2d, ones_hbm, zeros_hbm, n_pad, chunks_w):
    blk = n_pad // NS

    @functools.partial(
        pl.kernel,
        out_type=jax.ShapeDtypeStruct((NC, n_pad), jnp.float32),
        mesh=_sc_mesh(),
        compiler_params=_SC_PARAMS,
        scratch_types=[
            pltpu.VMEM((chunks_w, LANES), jnp.int32),
            pltpu.VMEM((LANES,), jnp.float32),
            pltpu.VMEM((blk,), jnp.float32),
            pltpu.VMEM_SHARED((n_pad,), jnp.float32),
        ],
    )
    def deg_kernel(r_hbm, one_hbm, z_hbm, out_hbm, ridx, ones_v, tmp, acc):
        c = lax.axis_index("c")
        s = lax.axis_index("s")
        w = s * NC + c
        pltpu.sync_copy(z_hbm.at[pl.ds(s * blk, blk)], tmp)
        pltpu.sync_copy(tmp, acc.at[pl.ds(s * blk, blk)])
        pltpu.sync_copy(one_hbm, ones_v)
        pltpu.sync_copy(r_hbm.at[pl.ds(w * chunks_w, chunks_w)], ridx)
        plsc.subcore_barrier()

        def body(j, carry):
            pltpu.sync_copy(ones_v, acc.at[ridx.at[j]], add=True)
            return carry

        lax.fori_loop(0, chunks_w, body, 0)
        plsc.subcore_barrier()
        pltpu.sync_copy(acc.at[pl.ds(s * blk, blk)], tmp)
        pltpu.sync_copy(tmp, out_hbm.at[c, pl.ds(s * blk, blk)])

    return deg_kernel(rows2d, ones_hbm, zeros_hbm)


def _spmm_sc(h, rows2d, cols2d, zeros_hbm, n_pad, feat, chunks_w):
    blk = n_pad // NS
    assert chunks_w % NBUF == 0

    @functools.partial(
        pl.kernel,
        out_type=jax.ShapeDtypeStruct((NC, n_pad, feat), jnp.float32),
        mesh=_sc_mesh(),
        compiler_params=_SC_PARAMS,
        scratch_types=[
            pltpu.VMEM((chunks_w + NBUF, LANES), jnp.int32),
            pltpu.VMEM((chunks_w, LANES), jnp.int32),
            [pltpu.VMEM((LANES, feat), jnp.float32) for _ in range(NBUF)],
            pltpu.VMEM((blk, feat), jnp.float32),
            pltpu.VMEM_SHARED((n_pad, feat), jnp.float32),
            [pltpu.SemaphoreType.DMA for _ in range(NBUF)],
        ],
    )
    def spmm_kernel(h_hbm, r_hbm, c_hbm, z_hbm, out_hbm,
                    ridx, cidx, msgs, tmp, acc, gsems):
        c = lax.axis_index("c")
        s = lax.axis_index("s")
        w = s * NC + c
        pltpu.sync_copy(z_hbm.at[pl.ds(s * blk, blk)], tmp)
        pltpu.sync_copy(tmp, acc.at[pl.ds(s * blk, blk)])
        pltpu.sync_copy(
            r_hbm.at[pl.ds(w * chunks_w, chunks_w)], ridx.at[pl.ds(0, chunks_w)]
        )
        pltpu.sync_copy(
            r_hbm.at[pl.ds(w * chunks_w, NBUF)], ridx.at[pl.ds(chunks_w, NBUF)]
        )
        pltpu.sync_copy(c_hbm.at[pl.ds(w * chunks_w, chunks_w)], cidx)
        plsc.subcore_barrier()

        for b in range(NBUF):
            pltpu.async_copy(h_hbm.at[ridx.at[b]], msgs[b], gsems[b])

        def body(t, carry):
            for b in range(NBUF):
                j = t * NBUF + b
                pltpu.make_async_copy(h_hbm.at[ridx.at[j]], msgs[b], gsems[b]).wait()
                pltpu.sync_copy(msgs[b], acc.at[cidx.at[j]], add=True)
                pltpu.async_copy(h_hbm.at[ridx.at[j + NBUF]], msgs[b], gsems[b])
            return carry

        lax.fori_loop(0, chunks_w // NBUF, body, 0)
        for b in range(NBUF):
            pltpu.make_async_copy(h_hbm.at[ridx.at[b]], msgs[b], gsems[b]).wait()
        plsc.subcore_barrier()
        pltpu.sync_copy(acc.at[pl.ds(s * blk, blk)], tmp)
        pltpu.sync_copy(tmp, out_hbm.at[c, pl.ds(s * blk, blk)])

    return spmm_kernel(h, rows2d, cols2d, zeros_hbm)


def _tc_layer1(deg0, deg1, xp, w1t, b1p, n_pad, f1):
    blkn = n_pad // 2
    d = xp.shape[1]

    def body(d0_ref, d1_ref, x_ref, w_ref, b_ref, h_ref, di_ref):
        di = lax.rsqrt(d0_ref[...] + d1_ref[...])
        h = jnp.dot(x_ref[...], w_ref[...], preferred_element_type=jnp.float32)
        h_ref[...] = (h + b_ref[...]) * di
        di_ref[...] = di

    return pl.pallas_call(
        body,
        grid=(pl.cdiv(n_pad, blkn),),
        in_specs=[
            pl.BlockSpec((blkn, 1), lambda i: (i, 0)),
            pl.BlockSpec((blkn, 1), lambda i: (i, 0)),
            pl.BlockSpec((blkn, d), lambda i: (i, 0)),
            pl.BlockSpec((d, f1), lambda i: (0, 0)),
            pl.BlockSpec((1, f1), lambda i: (0, 0)),
        ],
        out_specs=[
            pl.BlockSpec((blkn, f1), lambda i: (i, 0)),
            pl.BlockSpec((blkn, 1), lambda i: (i, 0)),
        ],
        out_shape=[
            jax.ShapeDtypeStruct((n_pad, f1), jnp.float32),
            jax.ShapeDtypeStruct((n_pad, 1), jnp.float32),
        ],
    )(deg0, deg1, xp, w1t, b1p)


def _tc_layer2(s1a, s1b, dinv, w2t, b2p, n_pad, f1, f2):
    blkn = n_pad // 2

    def body(a_ref, b_ref, di_ref, w_ref, bias_ref, out_ref):
        di = di_ref[...]
        h = jnp.maximum((a_ref[...] + b_ref[...]) * di, 0.0)
        out = jnp.dot(h, w_ref[...], preferred_element_type=jnp.float32)
        out_ref[...] = (out + bias_ref[...]) * di

    return pl.pallas_call(
        body,
        grid=(pl.cdiv(n_pad, blkn),),
        in_specs=[
            pl.BlockSpec((blkn, f1), lambda i: (i, 0)),
            pl.BlockSpec((blkn, f1), lambda i: (i, 0)),
            pl.BlockSpec((blkn, 1), lambda i: (i, 0)),
            pl.BlockSpec((f1, f2), lambda i: (0, 0)),
            pl.BlockSpec((1, f2), lambda i: (0, 0)),
        ],
        out_specs=pl.BlockSpec((blkn, f2), lambda i: (i, 0)),
        out_shape=jax.ShapeDtypeStruct((n_pad, f2), jnp.float32),
    )(s1a, s1b, dinv, w2t, b2p)


def _tc_logsoftmax(s2a, s2b, dinv, n, f2, ncls):
    blkn = 5056

    def body(a_ref, b_ref, di_ref, out_ref):
        z = ((a_ref[...] + b_ref[...]) * di_ref[...])[:, :ncls]
        m = jnp.max(z, axis=1, keepdims=True)
        e = jnp.exp(z - m)
        lse = jnp.log(jnp.sum(e, axis=1, keepdims=True)) + m
        out_ref[...] = z - lse

    return pl.pallas_call(
        body,
        grid=(pl.cdiv(n, blkn),),
        in_specs=[
            pl.BlockSpec((blkn, f2), lambda i: (i, 0)),
            pl.BlockSpec((blkn, f2), lambda i: (i, 0)),
            pl.BlockSpec((blkn, 1), lambda i: (i, 0)),
        ],
        out_specs=pl.BlockSpec((blkn, ncls), lambda i: (i, 0)),
        out_shape=jax.ShapeDtypeStruct((n, ncls), jnp.float32),
    )(s2a, s2b, dinv)


def kernel(x, edge_index, W1, b1, W2, b2):
    n, d = x.shape
    hid = W1.shape[0]
    ncls = W2.shape[0]
    e_tot = edge_index.shape[1] + n
    f1 = 16

    chunks_w = -(-e_tot // (LANES * NW))
    chunks_w = -(-chunks_w // 8) * 8
    e_pad = chunks_w * LANES * NW
    n_pad = -(-(n + 1) // LANES) * LANES

    ei = edge_index.astype(jnp.int32)
    appendix = jnp.asarray(np.concatenate([
        np.arange(n, dtype=np.int32),
        n + (np.arange(e_pad - e_tot, dtype=np.int32) % (n_pad - n)),
    ]))
    rows = jnp.concatenate([ei[0], appendix]).reshape(-1, LANES)
    cols = jnp.concatenate([ei[1], appendix]).reshape(-1, LANES)

    f2 = -(-ncls // 8) * 8
    ones_l = jnp.asarray(np.ones((LANES,), np.float32))
    z1 = jnp.asarray(np.zeros((n_pad,), np.float32))
    zf1 = jnp.asarray(np.zeros((n_pad, f1), np.float32))
    zf2 = jnp.asarray(np.zeros((n_pad, f2), np.float32))
    w1t = jnp.pad(W1.T, ((0, 0), (0, f1 - hid)))
    b1p = jnp.pad(b1, (0, f1 - hid)).reshape(1, f1)
    w2t = jnp.pad(W2.T, ((0, f1 - hid), (0, f2 - ncls)))
    b2p = jnp.pad(b2, (0, f2 - ncls)).reshape(1, f2)

    dega, degb = _deg_sc(rows, ones_l, z1, n_pad, chunks_w)
    deg0 = dega.reshape(n_pad, 1)
    deg1 = degb.reshape(n_pad, 1)

    h1p, dinv = _tc_layer1(deg0, deg1, x, w1t, b1p, n_pad, f1)
    s1 = _spmm_sc(h1p, rows, cols, zf1, n_pad, f1, chunks_w)
    h2p = _tc_layer2(s1[0], s1[1], dinv, w2t, b2p, n_pad, f1, f2)
    s2 = _spmm_sc(h2p, rows, cols, zf2, n_pad, f2, chunks_w)
    return _tc_logsoftmax(s2[0], s2[1], dinv, n, f2, ncls)


# speedup vs baseline: 45.5042x; 1.0087x over previous
import functools

import jax
import jax.numpy as jnp
import numpy as np
from jax import lax
from jax.experimental import pallas as pl
from jax.experimental.pallas import tpu as pltpu
from jax.experimental.pallas import tpu_sc as plsc

NC = 2
NS = 16
NW = NC * NS
LANES = 128
NBUF = 8


def _sc_mesh():
    return plsc.VectorSubcoreMesh(core_axis_name="c", subcore_axis_name="s")


_SC_PARAMS = pltpu.CompilerParams(use_tc_tiling_on_sc=False)


def _deg_sc(main_r, app_r, ones_hbm, zeros_hbm, n_pad, main_pw, app_pw):
    blk = n_pad // NS
    chunks_w = main_pw + app_pw

    @functools.partial(
        pl.kernel,
        out_type=jax.ShapeDtypeStruct((NC, n_pad), jnp.float32),
        mesh=_sc_mesh(),
        compiler_params=_SC_PARAMS,
        scratch_types=[
            pltpu.VMEM((chunks_w, LANES), jnp.int32),
            pltpu.VMEM((LANES,), jnp.float32),
            pltpu.VMEM((blk,), jnp.float32),
            pltpu.VMEM_SHARED((n_pad,), jnp.float32),
        ],
    )
    def deg_kernel(rm_hbm, ra_hbm, one_hbm, z_hbm, out_hbm, ridx, ones_v, tmp, acc):
        c = lax.axis_index("c")
        s = lax.axis_index("s")
        w = s * NC + c
        pltpu.sync_copy(z_hbm.at[pl.ds(s * blk, blk)], tmp)
        pltpu.sync_copy(tmp, acc.at[pl.ds(s * blk, blk)])
        pltpu.sync_copy(one_hbm, ones_v)
        pltpu.sync_copy(rm_hbm.at[pl.ds(w * main_pw, main_pw)],
                        ridx.at[pl.ds(0, main_pw)])
        pltpu.sync_copy(ra_hbm.at[pl.ds(w * app_pw, app_pw)],
                        ridx.at[pl.ds(main_pw, app_pw)])
        plsc.subcore_barrier()

        def body(j, carry):
            pltpu.sync_copy(ones_v, acc.at[ridx.at[j]], add=True)
            return carry

        lax.fori_loop(0, chunks_w, body, 0)
        plsc.subcore_barrier()
        pltpu.sync_copy(acc.at[pl.ds(s * blk, blk)], tmp)
        pltpu.sync_copy(tmp, out_hbm.at[c, pl.ds(s * blk, blk)])

    return deg_kernel(main_r, app_r, ones_hbm, zeros_hbm)


def _spmm_sc(h, main_r, app_r, main_c, app_c, zeros_hbm, n_pad, feat,
             main_pw, app_pw):
    blk = n_pad // NS
    chunks_w = main_pw + app_pw
    nbuf = NBUF
    assert chunks_w % nbuf == 0

    @functools.partial(
        pl.kernel,
        out_type=jax.ShapeDtypeStruct((NC, n_pad, feat), jnp.float32),
        mesh=_sc_mesh(),
        compiler_params=_SC_PARAMS,
        scratch_types=[
            pltpu.VMEM((chunks_w + nbuf, LANES), jnp.int32),
            pltpu.VMEM((chunks_w, LANES), jnp.int32),
            [pltpu.VMEM((LANES, feat), jnp.float32) for _ in range(nbuf)],
            pltpu.VMEM((blk, feat), jnp.float32),
            pltpu.VMEM_SHARED((n_pad, feat), jnp.float32),
            [pltpu.SemaphoreType.DMA for _ in range(nbuf)],
        ],
    )
    def spmm_kernel(h_hbm, rm_hbm, ra_hbm, cm_hbm, ca_hbm, z_hbm, out_hbm,
                    ridx, cidx, msgs, tmp, acc, gsems):
        c = lax.axis_index("c")
        s = lax.axis_index("s")
        w = s * NC + c
        pltpu.sync_copy(z_hbm.at[pl.ds(s * blk, blk)], tmp)
        pltpu.sync_copy(tmp, acc.at[pl.ds(s * blk, blk)])
        pltpu.sync_copy(rm_hbm.at[pl.ds(w * main_pw, main_pw)],
                        ridx.at[pl.ds(0, main_pw)])
        pltpu.sync_copy(ra_hbm.at[pl.ds(w * app_pw, app_pw)],
                        ridx.at[pl.ds(main_pw, app_pw)])
        pltpu.sync_copy(ra_hbm.at[pl.ds(0, nbuf)],
                        ridx.at[pl.ds(chunks_w, nbuf)])
        pltpu.sync_copy(cm_hbm.at[pl.ds(w * main_pw, main_pw)],
                        cidx.at[pl.ds(0, main_pw)])
        pltpu.sync_copy(ca_hbm.at[pl.ds(w * app_pw, app_pw)],
                        cidx.at[pl.ds(main_pw, app_pw)])
        plsc.subcore_barrier()

        for b in range(nbuf):
            pltpu.async_copy(h_hbm.at[ridx.at[b]], msgs[b], gsems[b])

        def body(t, carry):
            for b in range(nbuf):
                j = t * nbuf + b
                pltpu.make_async_copy(h_hbm.at[ridx.at[j]], msgs[b], gsems[b]).wait()
                pltpu.sync_copy(msgs[b], acc.at[cidx.at[j]], add=True)
                pltpu.async_copy(h_hbm.at[ridx.at[j + nbuf]], msgs[b], gsems[b])
            return carry

        lax.fori_loop(0, chunks_w // nbuf, body, 0)
        for b in range(nbuf):
            pltpu.make_async_copy(h_hbm.at[ridx.at[b]], msgs[b], gsems[b]).wait()
        plsc.subcore_barrier()
        pltpu.sync_copy(acc.at[pl.ds(s * blk, blk)], tmp)
        pltpu.sync_copy(tmp, out_hbm.at[c, pl.ds(s * blk, blk)])

    return spmm_kernel(h, main_r, app_r, main_c, app_c, zeros_hbm)


def _tc_layer1(deg0, deg1, xp, w1t, b1p, n_pad, f1):
    blkn = n_pad // 2
    d = xp.shape[1]

    def body(d0_ref, d1_ref, x_ref, w_ref, b_ref, h_ref, di_ref):
        di = lax.rsqrt(d0_ref[...] + d1_ref[...])
        h = jnp.dot(x_ref[...], w_ref[...], preferred_element_type=jnp.float32)
        h_ref[...] = (h + b_ref[...]) * di
        di_ref[...] = di

    return pl.pallas_call(
        body,
        grid=(pl.cdiv(n_pad, blkn),),
        in_specs=[
            pl.BlockSpec((blkn, 1), lambda i: (i, 0)),
            pl.BlockSpec((blkn, 1), lambda i: (i, 0)),
            pl.BlockSpec((blkn, d), lambda i: (i, 0)),
            pl.BlockSpec((d, f1), lambda i: (0, 0)),
            pl.BlockSpec((1, f1), lambda i: (0, 0)),
        ],
        out_specs=[
            pl.BlockSpec((blkn, f1), lambda i: (i, 0)),
            pl.BlockSpec((blkn, 1), lambda i: (i, 0)),
        ],
        out_shape=[
            jax.ShapeDtypeStruct((n_pad, f1), jnp.float32),
            jax.ShapeDtypeStruct((n_pad, 1), jnp.float32),
        ],
    )(deg0, deg1, xp, w1t, b1p)


def _tc_layer2(s1a, s1b, dinv, w2t, b2p, n_pad, f1, f2):
    blkn = n_pad // 2

    def body(a_ref, b_ref, di_ref, w_ref, bias_ref, out_ref):
        di = di_ref[...]
        h = jnp.maximum((a_ref[...] + b_ref[...]) * di, 0.0)
        out = jnp.dot(h, w_ref[...], preferred_element_type=jnp.float32)
        out_ref[...] = (out + bias_ref[...]) * di

    return pl.pallas_call(
        body,
        grid=(pl.cdiv(n_pad, blkn),),
        in_specs=[
            pl.BlockSpec((blkn, f1), lambda i: (i, 0)),
            pl.BlockSpec((blkn, f1), lambda i: (i, 0)),
            pl.BlockSpec((blkn, 1), lambda i: (i, 0)),
            pl.BlockSpec((f1, f2), lambda i: (0, 0)),
            pl.BlockSpec((1, f2), lambda i: (0, 0)),
        ],
        out_specs=pl.BlockSpec((blkn, f2), lambda i: (i, 0)),
        out_shape=jax.ShapeDtypeStruct((n_pad, f2), jnp.float32),
    )(s1a, s1b, dinv, w2t, b2p)


def _tc_logsoftmax(s2a, s2b, dinv, n, f2, ncls):
    blkn = 5056

    def body(a_ref, b_ref, di_ref, out_ref):
        z = ((a_ref[...] + b_ref[...]) * di_ref[...])[:, :ncls]
        m = jnp.max(z, axis=1, keepdims=True)
        e = jnp.exp(z - m)
        lse = jnp.log(jnp.sum(e, axis=1, keepdims=True)) + m
        out_ref[...] = z - lse

    return pl.pallas_call(
        body,
        grid=(pl.cdiv(n, blkn),),
        in_specs=[
            pl.BlockSpec((blkn, f2), lambda i: (i, 0)),
            pl.BlockSpec((blkn, f2), lambda i: (i, 0)),
            pl.BlockSpec((blkn, 1), lambda i: (i, 0)),
        ],
        out_specs=pl.BlockSpec((blkn, ncls), lambda i: (i, 0)),
        out_shape=jax.ShapeDtypeStruct((n, ncls), jnp.float32),
    )(s2a, s2b, dinv)


def kernel(x, edge_index, W1, b1, W2, b2):
    n, d = x.shape
    hid = W1.shape[0]
    ncls = W2.shape[0]
    e_tot = edge_index.shape[1] + n
    f1 = 16

    n_pad = -(-(n + 1) // LANES) * LANES
    e_main = edge_index.shape[1]

    main_pw = e_main // (LANES * NW)
    main_len = main_pw * LANES * NW
    leftover = e_main - main_len
    app_pw = -(-(leftover + n) // (LANES * NW))
    app_pw += (-(main_pw + app_pw)) % NBUF
    app_real = leftover + n
    app_len = app_pw * LANES * NW
    appendix_lit = jnp.asarray(
        np.concatenate([
            np.arange(n, dtype=np.int32),
            n + (np.arange(app_len - app_real, dtype=np.int32) % (n_pad - n)),
        ])
    )

    ei = edge_index.astype(jnp.int32)
    main_r = ei[0, :main_len].reshape(-1, LANES)
    main_c = ei[1, :main_len].reshape(-1, LANES)
    app_r = jnp.concatenate([ei[0, main_len:], appendix_lit]).reshape(-1, LANES)
    app_c = jnp.concatenate([ei[1, main_len:], appendix_lit]).reshape(-1, LANES)

    f2 = -(-ncls // 8) * 8
    ones_l = jnp.asarray(np.ones((LANES,), np.float32))
    z1 = jnp.asarray(np.zeros((n_pad,), np.float32))
    zf1 = jnp.asarray(np.zeros((n_pad, f1), np.float32))
    zf2 = jnp.asarray(np.zeros((n_pad, f2), np.float32))
    w1t = jnp.pad(W1.T, ((0, 0), (0, f1 - hid)))
    b1p = jnp.pad(b1, (0, f1 - hid)).reshape(1, f1)
    w2t = jnp.pad(W2.T, ((0, f1 - hid), (0, f2 - ncls)))
    b2p = jnp.pad(b2, (0, f2 - ncls)).reshape(1, f2)

    dega, degb = _deg_sc(main_r, app_r, ones_l, z1, n_pad, main_pw, app_pw)
    deg0 = dega.reshape(n_pad, 1)
    deg1 = degb.reshape(n_pad, 1)

    h1p, dinv = _tc_layer1(deg0, deg1, x, w1t, b1p, n_pad, f1)
    s1 = _spmm_sc(h1p, main_r, app_r, main_c, app_c, zf1, n_pad, f1,
                  main_pw, app_pw)
    h2p = _tc_layer2(s1[0], s1[1], dinv, w2t, b2p, n_pad, f1, f2)
    s2 = _spmm_sc(h2p, main_r, app_r, main_c, app_c, zf2, n_pad, f2,
                  main_pw, app_pw)
    return _tc_logsoftmax(s2[0], s2[1], dinv, n, f2, ncls)
